# baseline (speedup 1.0000x reference)
import functools

import jax
import jax.numpy as jnp
from jax import lax
from jax.experimental import pallas as pl
from jax.experimental.pallas import tpu as pltpu
from jax.experimental.pallas import tpu_sc as plsc

B = 16384
F = 26
V = 100000
D = 32
ND = 13
H = 64

NC = 2
NS = 16
NW = NC * NS

ROWS = B * F
RPW = ROWS // NW
CH = 1024
NCH = RPW // CH
GPC = CH // 128

_BN_INV = 1.0 / (1.0 + 1e-5) ** 0.5


def _sc_gather(table2d, idx4d):
    mesh = plsc.VectorSubcoreMesh(core_axis_name="c", subcore_axis_name="s")

    @functools.partial(
        pl.kernel,
        out_type=jax.ShapeDtypeStruct((ROWS, D), jnp.float32),
        mesh=mesh,
        scratch_types=[
            pltpu.VMEM((NCH, GPC, 128), jnp.int32),
            pltpu.VMEM((CH, D), jnp.float32),
            pltpu.SemaphoreType.DMA,
        ],
        compiler_params=pltpu.CompilerParams(use_tc_tiling_on_sc=False),
    )
    def gather_kernel(table_hbm, idx_hbm, out_hbm, idx_v, rows_v, sem):
        wid = lax.axis_index("s") * NC + lax.axis_index("c")
        base = wid * RPW
        pltpu.sync_copy(idx_hbm.at[wid], idx_v)

        @pl.loop(0, NCH)
        def _chunk(c):
            copies = [
                pltpu.async_copy(
                    table_hbm.at[idx_v.at[c, j]],
                    rows_v.at[pl.ds(j * 128, 128)],
                    sem,
                )
                for j in range(GPC)
            ]
            for cp in copies:
                cp.wait()
            pltpu.sync_copy(rows_v, out_hbm.at[pl.ds(base + c * CH, CH)])

    return gather_kernel(table2d, idx4d)


def _mlp_body(emb_ref, num_ref, w1a_ref, w1b_ref, b1_ref, w2_ref, b2_ref,
              bnw_ref, bnb_ref, out_ref):
    x = emb_ref[...]
    h = jnp.dot(x, w1a_ref[...], preferred_element_type=jnp.float32)
    num_n = num_ref[...] * (bnw_ref[...] * _BN_INV) + bnb_ref[...]
    h = h + jnp.dot(num_n, w1b_ref[...], preferred_element_type=jnp.float32)
    h = jnp.maximum(h + b1_ref[...], 0.0)
    out_ref[...] = jnp.dot(h, w2_ref[...], preferred_element_type=jnp.float32) + b2_ref[...]


def _mlp(emb_concat, num, W1, b1, W2, b2, bn_w, bn_b):
    BM = 1024
    W1a = W1[: F * D]
    W1b = W1[F * D :]
    return pl.pallas_call(
        _mlp_body,
        grid=(B // BM,),
        in_specs=[
            pl.BlockSpec((BM, F * D), lambda i: (i, 0)),
            pl.BlockSpec((BM, ND), lambda i: (i, 0)),
            pl.BlockSpec((F * D, H), lambda i: (0, 0)),
            pl.BlockSpec((ND, H), lambda i: (0, 0)),
            pl.BlockSpec((1, H), lambda i: (0, 0)),
            pl.BlockSpec((H, 1), lambda i: (0, 0)),
            pl.BlockSpec((1, 1), lambda i: (0, 0)),
            pl.BlockSpec((1, ND), lambda i: (0, 0)),
            pl.BlockSpec((1, ND), lambda i: (0, 0)),
        ],
        out_specs=pl.BlockSpec((BM, 1), lambda i: (i, 0)),
        out_shape=jax.ShapeDtypeStruct((B, 1), jnp.float32),
    )(emb_concat, num, W1a, W1b, b1.reshape(1, H), W2, b2.reshape(1, 1),
      bn_w.reshape(1, ND), bn_b.reshape(1, ND))


def kernel(cat_idx, num, emb_tables, W1, b1, W2, b2, bn_w, bn_b):
    table2d = emb_tables.reshape(F * V, D)
    flat_idx = (
        cat_idx.astype(jnp.int32) + (jnp.arange(F, dtype=jnp.int32) * V)[None, :]
    ).reshape(NW, NCH, GPC, 128)
    emb_flat = _sc_gather(table2d, flat_idx)
    emb_concat = emb_flat.reshape(B, F * D)
    out = _mlp(emb_concat, num, W1, b1, W2, b2, bn_w, bn_b)
    return (out, emb_concat)


# speedup vs baseline: 1.0066x; 1.0066x over previous
import functools

import jax
import jax.numpy as jnp
from jax import lax
from jax.experimental import pallas as pl
from jax.experimental.pallas import tpu as pltpu
from jax.experimental.pallas import tpu_sc as plsc

B = 16384
F = 26
V = 100000
D = 32
ND = 13
H = 64

NC = 2
NS = 16
NW = NC * NS

BPW = B // NW
GL = 64
BG = BPW // GL

_BN_INV = 1.0 / (1.0 + 1e-5) ** 0.5


def _sc_gather(emb_tables, idx4d):
    mesh = plsc.VectorSubcoreMesh(core_axis_name="c", subcore_axis_name="s")

    @functools.partial(
        pl.kernel,
        out_type=jax.ShapeDtypeStruct((B, F * D), jnp.float32),
        mesh=mesh,
        scratch_types=[
            pltpu.VMEM((BG, F, GL), jnp.int32),
            pltpu.VMEM((2, F, GL, D), jnp.float32),
            pltpu.SemaphoreType.DMA,
            pltpu.SemaphoreType.DMA,
        ],
        compiler_params=pltpu.CompilerParams(use_tc_tiling_on_sc=False),
    )
    def gather_kernel(table_hbm, idx_hbm, out_hbm, idx_v, buf_v, gsem, wsem):
        wid = lax.axis_index("s") * NC + lax.axis_index("c")
        base = wid * BPW
        pltpu.sync_copy(idx_hbm.at[wid], idx_v)

        def fire_gathers(g, p):
            @pl.loop(0, F)
            def _f(f):
                pltpu.async_copy(
                    table_hbm.at[f].at[idx_v.at[g, f]],
                    buf_v.at[p, f],
                    gsem,
                )

        def drain_gathers(p):
            @pl.loop(0, F)
            def _f(f):
                pltpu.make_async_copy(
                    table_hbm.at[0].at[idx_v.at[0, 0]], buf_v.at[p, 0], gsem
                ).wait()

        def fire_writes(g, p):
            b0 = base + g * GL

            @pl.loop(0, F)
            def _f(f):
                pltpu.async_copy(
                    buf_v.at[p, f],
                    out_hbm.at[pl.ds(b0, GL), pl.ds(f * D, D)],
                    wsem,
                )

        def drain_writes(p):
            @pl.loop(0, F)
            def _f(f):
                pltpu.make_async_copy(
                    buf_v.at[p, 0],
                    out_hbm.at[pl.ds(base, GL), pl.ds(0, D)],
                    wsem,
                ).wait()

        fire_gathers(0, 0)
        drain_gathers(0)
        fire_gathers(1, 1)
        fire_writes(0, 0)

        @pl.loop(2, BG, step=2)
        def _g(g):
            drain_gathers(1)
            drain_writes(0)
            fire_gathers(g, 0)
            fire_writes(g - 1, 1)
            drain_gathers(0)
            drain_writes(1)
            fire_gathers(g + 1, 1)
            fire_writes(g, 0)

        drain_gathers(1)
        drain_writes(0)
        fire_writes(BG - 1, 1)
        drain_writes(1)

    return gather_kernel(emb_tables, idx4d)


def _mlp_body(emb_ref, num_ref, w1a_ref, w1b_ref, b1_ref, w2_ref, b2_ref,
              bnw_ref, bnb_ref, out_ref):
    x = emb_ref[...]
    h = jnp.dot(x, w1a_ref[...], preferred_element_type=jnp.float32)
    num_n = num_ref[...] * (bnw_ref[...] * _BN_INV) + bnb_ref[...]
    h = h + jnp.dot(num_n, w1b_ref[...], preferred_element_type=jnp.float32)
    h = jnp.maximum(h + b1_ref[...], 0.0)
    out_ref[...] = jnp.dot(h, w2_ref[...], preferred_element_type=jnp.float32) + b2_ref[...]


def _mlp(emb_concat, num, W1, b1, W2, b2, bn_w, bn_b):
    BM = 1024
    W1a = W1[: F * D]
    W1b = W1[F * D :]
    return pl.pallas_call(
        _mlp_body,
        grid=(B // BM,),
        in_specs=[
            pl.BlockSpec((BM, F * D), lambda i: (i, 0)),
            pl.BlockSpec((BM, ND), lambda i: (i, 0)),
            pl.BlockSpec((F * D, H), lambda i: (0, 0)),
            pl.BlockSpec((ND, H), lambda i: (0, 0)),
            pl.BlockSpec((1, H), lambda i: (0, 0)),
            pl.BlockSpec((H, 1), lambda i: (0, 0)),
            pl.BlockSpec((1, 1), lambda i: (0, 0)),
            pl.BlockSpec((1, ND), lambda i: (0, 0)),
            pl.BlockSpec((1, ND), lambda i: (0, 0)),
        ],
        out_specs=pl.BlockSpec((BM, 1), lambda i: (i, 0)),
        out_shape=jax.ShapeDtypeStruct((B, 1), jnp.float32),
    )(emb_concat, num, W1a, W1b, b1.reshape(1, H), W2, b2.reshape(1, 1),
      bn_w.reshape(1, ND), bn_b.reshape(1, ND))


def kernel(cat_idx, num, emb_tables, W1, b1, W2, b2, bn_w, bn_b):
    idx4d = cat_idx.astype(jnp.int32).reshape(NW, BG, GL, F).transpose(0, 1, 3, 2)
    emb_concat = _sc_gather(emb_tables, idx4d)
    out = _mlp(emb_concat, num, W1, b1, W2, b2, bn_w, bn_b)
    return (out, emb_concat)
